# baseline (speedup 1.0000x reference)
import functools

import jax
import jax.numpy as jnp
from jax import lax
from jax.experimental import pallas as pl
from jax.experimental.pallas import tpu as pltpu
from jax.experimental.pallas import tpu_sc as plsc

_GROUPS = [(0, 'hp'), (1, 'stat'), (2, 'stat'), (3, 'stat'), (4, 'stat'),
           (5, 'stat'), (6, 'stat'), (7, 'power'), (8, 'power'), (9, 'power'),
           (10, 'power'), (11, 'damage'), (12, 'damage'), (13, 'damage'),
           (14, 'damage'), (15, 'turn'), (16, 'rating'), (17, 'rating')]
_CFG = {'hp': (1.0, 100, 16), 'stat': (600.0, 600, 16), 'power': (250.0, 250, 16),
        'damage': (600.0, 600, 4), 'turn': (40.0, 40, 16), 'rating': (2000.0, 100, 16)}
_BANK_ORDER = ['hp', 'stat', 'power', 'damage', 'turn', 'rating']

_BASES = {}
_off = 0
for _name in _BANK_ORDER:
    _BASES[_name] = _off
    _maxv, _nbins, _w = _CFG[_name]
    _off += (_nbins + 1) * (_w + 1)
_BANK_WORDS = _off
_BANK_PAD = (-_BANK_WORDS) % 16
_BANK_TOTAL = _BANK_WORDS + _BANK_PAD

_FEATS = []
for _xcol, _name in _GROUPS:
    _maxv, _nbins, _w = _CFG[_name]
    _FEATS.append((_xcol, _BASES[_name], _w + 1, _w, _maxv, _nbins))

_COLPLAN = []
for _fi, (_xcol, _b, _stride, _w, _mv, _nb) in enumerate(_FEATS):
    for _o in range(_w):
        _COLPLAN.append((_fi, _o))
_N_EMB = len(_COLPLAN)

_D_IN = 64
_D_OUT = _N_EMB + (_D_IN - len(_FEATS))
_NC, _NS = 2, 16
_NW = _NC * _NS
_CHB = 4
_L = 16


def _splat(v):
    return jnp.full((_L,), v, jnp.int32)


def _tec_body(x_hbm, banks_hbm, out_hbm, xv, banksv, outv, *, bsz, seq):
    rows_per = bsz // _NW
    n_ch = rows_per // _CHB
    pos_per_ch = _CHB * seq
    n_groups = (pos_per_ch + _L - 1) // _L
    wid = lax.axis_index("s") * _NC + lax.axis_index("c")
    base_b = wid * rows_per

    pltpu.sync_copy(banks_hbm, banksv)

    iota = lax.iota(jnp.int32, _L)

    def chunk_body(g, carry):
        b0 = base_b + g * _CHB
        for r in range(_CHB):
            pltpu.sync_copy(x_hbm.at[b0 + r], xv.at[pl.ds(r * seq, seq)])

        @plsc.parallel_loop(0, n_groups)
        def group_body(t):
            pv_raw = t * _L + iota
            pv = jnp.minimum(pv_raw, pos_per_ch - 1)
            mask = pv_raw < pos_per_ch
            rowaddr = []
            for (xcol, bank_base, stride, w, maxv, nbins) in _FEATS:
                raw = plsc.load_gather(xv, [pv, _splat(xcol)])
                clamped = jnp.clip(raw, 0.0, maxv)
                b = ((clamped / maxv) * nbins).astype(jnp.int32)
                b = jnp.clip(b, 0, nbins)
                rowaddr.append(bank_base + b * stride)

            def col_val(col):
                if col < _N_EMB:
                    fi, off = _COLPLAN[col]
                    return plsc.load_gather(banksv, [rowaddr[fi] + off])
                return plsc.load_gather(xv, [pv, _splat(col - _N_EMB + len(_FEATS))])

            _K = 8
            for lo in range(0, _D_OUT, _K):
                batch = range(lo, min(lo + _K, _D_OUT))
                vals = [col_val(col) for col in batch]
                for col, val in zip(batch, vals):
                    plsc.store_scatter(outv, [pv, _splat(col)], val, mask=mask)

        for r in range(_CHB):
            pltpu.sync_copy(outv.at[pl.ds(r * seq, seq)], out_hbm.at[b0 + r])
        return carry

    lax.fori_loop(0, n_ch, chunk_body, 0)


def kernel(x, hp_bank, stat_bank, power_bank, damage_bank, turn_bank,
           rating_bank, group_idx):
    bsz, seq, d_in = x.shape

    def _padrow(b):
        return jnp.pad(b, ((0, 0), (0, 1))).reshape(-1)

    banks_flat = jnp.concatenate([
        _padrow(hp_bank), _padrow(stat_bank), _padrow(power_bank),
        _padrow(damage_bank), _padrow(turn_bank), _padrow(rating_bank),
        jnp.zeros((_BANK_PAD,), jnp.float32)])

    mesh = plsc.VectorSubcoreMesh(core_axis_name="c", subcore_axis_name="s")
    run = functools.partial(
        pl.kernel,
        mesh=mesh,
        compiler_params=pltpu.CompilerParams(
            needs_layout_passes=False, use_tc_tiling_on_sc=False),
        out_type=jax.ShapeDtypeStruct((bsz, seq, _D_OUT), jnp.float32),
        scratch_types=[
            pltpu.VMEM((_CHB * seq, _D_IN), jnp.float32),
            pltpu.VMEM((_BANK_TOTAL,), jnp.float32),
            pltpu.VMEM((_CHB * seq, _D_OUT), jnp.float32),
        ],
    )(functools.partial(_tec_body, bsz=bsz, seq=seq))
    return run(x, banks_flat)


# speedup vs baseline: 1.0008x; 1.0008x over previous
import functools

import jax
import jax.numpy as jnp
from jax import lax
from jax.experimental import pallas as pl
from jax.experimental.pallas import tpu as pltpu
from jax.experimental.pallas import tpu_sc as plsc

_GROUPS = [(0, 'hp'), (1, 'stat'), (2, 'stat'), (3, 'stat'), (4, 'stat'),
           (5, 'stat'), (6, 'stat'), (7, 'power'), (8, 'power'), (9, 'power'),
           (10, 'power'), (11, 'damage'), (12, 'damage'), (13, 'damage'),
           (14, 'damage'), (15, 'turn'), (16, 'rating'), (17, 'rating')]
_CFG = {'hp': (1.0, 100, 16), 'stat': (600.0, 600, 16), 'power': (250.0, 250, 16),
        'damage': (600.0, 600, 4), 'turn': (40.0, 40, 16), 'rating': (2000.0, 100, 16)}
_BANK_ORDER = ['hp', 'stat', 'power', 'damage', 'turn', 'rating']

_BASES = {}
_off = 0
for _name in _BANK_ORDER:
    _BASES[_name] = _off
    _maxv, _nbins, _w = _CFG[_name]
    _off += (_nbins + 1) * (_w + 1)
_BANK_WORDS = _off
_BANK_PAD = (-_BANK_WORDS) % 16
_BANK_TOTAL = _BANK_WORDS + _BANK_PAD

_FEATS = []
for _xcol, _name in _GROUPS:
    _maxv, _nbins, _w = _CFG[_name]
    _FEATS.append((_xcol, _BASES[_name], _w + 1, _w, _maxv, _nbins))

_COLPLAN = []
for _fi, (_xcol, _b, _stride, _w, _mv, _nb) in enumerate(_FEATS):
    for _o in range(_w):
        _COLPLAN.append((_fi, _o))
_N_EMB = len(_COLPLAN)

_D_IN = 64
_D_OUT = _N_EMB + (_D_IN - len(_FEATS))
_NC, _NS = 2, 16
_NW = _NC * _NS
_CHB = 4
_L = 16


def _splat(v):
    return jnp.full((_L,), v, jnp.int32)


def _tec_body(x_hbm, banks_hbm, out_hbm, xv, banksv, outv, *, bsz, seq):
    rows_per = bsz // _NW
    n_ch = rows_per // _CHB
    pos_per_ch = _CHB * seq
    n_groups = (pos_per_ch + _L - 1) // _L
    wid = lax.axis_index("s") * _NC + lax.axis_index("c")
    base_b = wid * rows_per

    pltpu.sync_copy(banks_hbm, banksv)

    iota = lax.iota(jnp.int32, _L)

    def chunk_body(g, carry):
        b0 = base_b + g * _CHB
        for r in range(_CHB):
            pltpu.sync_copy(x_hbm.at[b0 + r], xv.at[pl.ds(r * seq, seq)])

        @plsc.parallel_loop(0, n_groups)
        def group_body(t):
            pv_raw = t * _L + iota
            pv = jnp.minimum(pv_raw, pos_per_ch - 1)
            mask = pv_raw < pos_per_ch
            rowaddr = []
            for (xcol, bank_base, stride, w, maxv, nbins) in _FEATS:
                raw = plsc.load_gather(xv, [pv, _splat(xcol)])
                clamped = jnp.clip(raw, 0.0, maxv)
                b = ((clamped / maxv) * nbins).astype(jnp.int32)
                b = jnp.clip(b, 0, nbins)
                rowaddr.append(bank_base + b * stride)

            def col_val(col):
                if col < _N_EMB:
                    fi, off = _COLPLAN[col]
                    return plsc.load_gather(banksv, [rowaddr[fi] + off])
                return plsc.load_gather(xv, [pv, _splat(col - _N_EMB + len(_FEATS))])

            _K = 8
            for lo in range(0, _D_OUT, _K):
                batch = range(lo, min(lo + _K, _D_OUT))
                vals = [col_val(col) for col in batch]
                for col, val in zip(batch, vals):
                    plsc.store_scatter(outv, [pv, _splat(col)], val, mask=mask)

        for r in range(_CHB):
            pltpu.sync_copy(outv.at[pl.ds(r * seq, seq)], out_hbm.at[b0 + r])
        return carry

    lax.fori_loop(0, n_ch, chunk_body, 0)


def kernel(x, hp_bank, stat_bank, power_bank, damage_bank, turn_bank,
           rating_bank, group_idx):
    bsz, seq, d_in = x.shape

    def _padrow(b):
        return jnp.pad(b, ((0, 0), (0, 1))).reshape(-1)

    banks_flat = jnp.concatenate([
        _padrow(hp_bank), _padrow(stat_bank), _padrow(power_bank),
        _padrow(damage_bank), _padrow(turn_bank), _padrow(rating_bank),
        jnp.zeros((_BANK_PAD,), jnp.float32)])

    mesh = plsc.VectorSubcoreMesh(core_axis_name="c", subcore_axis_name="s")
    run = functools.partial(
        pl.kernel,
        mesh=mesh,
        compiler_params=pltpu.CompilerParams(
            needs_layout_passes=False, use_tc_tiling_on_sc=False,
            disable_bounds_checks=True),
        out_type=jax.ShapeDtypeStruct((bsz, seq, _D_OUT), jnp.float32),
        scratch_types=[
            pltpu.VMEM((_CHB * seq, _D_IN), jnp.float32),
            pltpu.VMEM((_BANK_TOTAL,), jnp.float32),
            pltpu.VMEM((_CHB * seq, _D_OUT), jnp.float32),
        ],
    )(functools.partial(_tec_body, bsz=bsz, seq=seq))
    return run(x, banks_flat)


# speedup vs baseline: 1.0322x; 1.0314x over previous
import functools

import jax
import jax.numpy as jnp
from jax import lax
from jax.experimental import pallas as pl
from jax.experimental.pallas import tpu as pltpu
from jax.experimental.pallas import tpu_sc as plsc

_GROUPS = [(0, 'hp'), (1, 'stat'), (2, 'stat'), (3, 'stat'), (4, 'stat'),
           (5, 'stat'), (6, 'stat'), (7, 'power'), (8, 'power'), (9, 'power'),
           (10, 'power'), (11, 'damage'), (12, 'damage'), (13, 'damage'),
           (14, 'damage'), (15, 'turn'), (16, 'rating'), (17, 'rating')]
_CFG = {'hp': (1.0, 100, 16), 'stat': (600.0, 600, 16), 'power': (250.0, 250, 16),
        'damage': (600.0, 600, 4), 'turn': (40.0, 40, 16), 'rating': (2000.0, 100, 16)}
_BANK_ORDER = ['hp', 'stat', 'power', 'damage', 'turn', 'rating']

_BASES = {}
_off = 0
for _name in _BANK_ORDER:
    _BASES[_name] = _off
    _maxv, _nbins, _w = _CFG[_name]
    _off += (_nbins + 1) * (_w + 1)
_BANK_WORDS = _off
_BANK_PAD = (-_BANK_WORDS) % 16
_BANK_TOTAL = _BANK_WORDS + _BANK_PAD

_FEATS = []
for _xcol, _name in _GROUPS:
    _maxv, _nbins, _w = _CFG[_name]
    _FEATS.append((_xcol, _BASES[_name], _w + 1, _w, _maxv, _nbins))

_COLPLAN = []
for _fi, (_xcol, _b, _stride, _w, _mv, _nb) in enumerate(_FEATS):
    for _o in range(_w):
        _COLPLAN.append((_fi, _o))
_N_EMB = len(_COLPLAN)

_D_IN = 64
_D_OUT = _N_EMB + (_D_IN - len(_FEATS))
_NC, _NS = 2, 16
_NW = _NC * _NS
_CHB = 4
_L = 16


def _splat(v):
    return jnp.full((_L,), v, jnp.int32)


def _tec_body(x_hbm, banks_hbm, out_hbm, xv, banksv, outv, sem_in, sem_out, *, bsz, seq):
    rows_per = bsz // _NW
    n_ch = rows_per // _CHB
    pos_per_ch = _CHB * seq
    n_groups = (pos_per_ch + _L - 1) // _L
    wid = lax.axis_index("s") * _NC + lax.axis_index("c")
    base_b = wid * rows_per

    pltpu.sync_copy(banks_hbm, banksv)

    iota = lax.iota(jnp.int32, _L)

    def chunk_body(g, carry):
        b0 = base_b + g * _CHB
        cps = [pltpu.async_copy(x_hbm.at[b0 + r], xv.at[pl.ds(r * seq, seq)], sem_in)
               for r in range(_CHB)]
        for cp in cps:
            cp.wait()

        @plsc.parallel_loop(0, n_groups)
        def group_body(t):
            pv_raw = t * _L + iota
            pv = jnp.minimum(pv_raw, pos_per_ch - 1)
            mask = pv_raw < pos_per_ch
            rowaddr = []
            for (xcol, bank_base, stride, w, maxv, nbins) in _FEATS:
                raw = plsc.load_gather(xv, [pv, _splat(xcol)])
                clamped = jnp.clip(raw, 0.0, maxv)
                b = ((clamped / maxv) * nbins).astype(jnp.int32)
                b = jnp.clip(b, 0, nbins)
                rowaddr.append(bank_base + b * stride)

            def col_val(col):
                if col < _N_EMB:
                    fi, off = _COLPLAN[col]
                    return plsc.load_gather(banksv, [rowaddr[fi] + off])
                return plsc.load_gather(xv, [pv, _splat(col - _N_EMB + len(_FEATS))])

            _K = 8
            for lo in range(0, _D_OUT, _K):
                batch = range(lo, min(lo + _K, _D_OUT))
                vals = [col_val(col) for col in batch]
                for col, val in zip(batch, vals):
                    plsc.store_scatter(outv, [pv, _splat(col)], val, mask=mask)

        cps = [pltpu.async_copy(outv.at[pl.ds(r * seq, seq)], out_hbm.at[b0 + r], sem_out)
               for r in range(_CHB)]
        for cp in cps:
            cp.wait()
        return carry

    lax.fori_loop(0, n_ch, chunk_body, 0)


def kernel(x, hp_bank, stat_bank, power_bank, damage_bank, turn_bank,
           rating_bank, group_idx):
    bsz, seq, d_in = x.shape

    def _padrow(b):
        return jnp.pad(b, ((0, 0), (0, 1))).reshape(-1)

    banks_flat = jnp.concatenate([
        _padrow(hp_bank), _padrow(stat_bank), _padrow(power_bank),
        _padrow(damage_bank), _padrow(turn_bank), _padrow(rating_bank),
        jnp.zeros((_BANK_PAD,), jnp.float32)])

    mesh = plsc.VectorSubcoreMesh(core_axis_name="c", subcore_axis_name="s")
    run = functools.partial(
        pl.kernel,
        mesh=mesh,
        compiler_params=pltpu.CompilerParams(
            needs_layout_passes=False, use_tc_tiling_on_sc=False,
            disable_bounds_checks=True),
        out_type=jax.ShapeDtypeStruct((bsz, seq, _D_OUT), jnp.float32),
        scratch_types=[
            pltpu.VMEM((_CHB * seq, _D_IN), jnp.float32),
            pltpu.VMEM((_BANK_TOTAL,), jnp.float32),
            pltpu.VMEM((_CHB * seq, _D_OUT), jnp.float32),
            pltpu.SemaphoreType.DMA,
            pltpu.SemaphoreType.DMA,
        ],
    )(functools.partial(_tec_body, bsz=bsz, seq=seq))
    return run(x, banks_flat)
